# baseline (speedup 1.0000x reference)
import jax
import jax.numpy as jnp
from jax import lax
from jax.experimental import pallas as pl
from jax.experimental.pallas import tpu as pltpu
from jax.experimental.pallas import tpu_sc as plsc

_NC = 2
_NS = 16
_NW = _NC * _NS

_CHUNK = 128


def _emb_body(item_idx, user_idx, table_item, table_user, out_item, out_user,
              idx_v, rows_v, uidx_v, urows_v, sem):
    wid = lax.axis_index("s") * _NC + lax.axis_index("c")
    per_w = idx_v.shape[0]
    n_chunks = per_w // _CHUNK

    pltpu.sync_copy(item_idx.at[pl.ds(wid * per_w, per_w)], idx_v)

    def chunk_step(j, carry):
        pltpu.async_copy(
            table_item.at[idx_v.at[pl.ds(j * _CHUNK, _CHUNK)]], rows_v, sem
        ).wait()
        base = wid * per_w + j * _CHUNK
        pltpu.sync_copy(rows_v, out_item.at[pl.ds(base, _CHUNK)])
        return carry

    lax.fori_loop(0, n_chunks, chunk_step, 0)

    per_w_user = uidx_v.shape[0]
    pltpu.sync_copy(user_idx.at[pl.ds(wid * per_w_user, per_w_user)], uidx_v)
    pltpu.async_copy(table_user.at[uidx_v], urows_v, sem).wait()
    pltpu.sync_copy(urows_v, out_user.at[pl.ds(wid * per_w_user, per_w_user)])


def kernel(values_item_hist, values_user_cat, table_item, table_user):
    n_hist = values_item_hist.shape[0]
    n_user = values_user_cat.shape[0]
    dim_item = table_item.shape[1]
    dim_user = table_user.shape[1]

    per_w = n_hist // _NW
    per_w_user = n_user // _NW

    mesh = plsc.VectorSubcoreMesh(core_axis_name="c", subcore_axis_name="s")
    f = pl.kernel(
        _emb_body,
        out_type=(
            jax.ShapeDtypeStruct((n_hist, dim_item), table_item.dtype),
            jax.ShapeDtypeStruct((n_user, dim_user), table_user.dtype),
        ),
        mesh=mesh,
        compiler_params=pltpu.CompilerParams(use_tc_tiling_on_sc=False),
        scratch_types=[
            pltpu.VMEM((per_w,), jnp.int32),
            pltpu.VMEM((_CHUNK, dim_item), jnp.float32),
            pltpu.VMEM((per_w_user,), jnp.int32),
            pltpu.VMEM((per_w_user, dim_user), jnp.float32),
            pltpu.SemaphoreType.DMA,
        ],
    )
    return f(values_item_hist, values_user_cat, table_item, table_user)


# speedup vs baseline: 1.0439x; 1.0439x over previous
import jax
import jax.numpy as jnp
from jax import lax
from jax.experimental import pallas as pl
from jax.experimental.pallas import tpu as pltpu
from jax.experimental.pallas import tpu_sc as plsc

_NC = 2
_NS = 16
_NW = _NC * _NS

_CHUNK = 128
_NBUF = 10


def _emb_body(item_idx, user_idx, table_item, table_user, out_item, out_user,
              idx_v, rows_v, uidx_v, urows_v, gsem, osem, usem):
    wid = lax.axis_index("s") * _NC + lax.axis_index("c")
    per_w = idx_v.shape[0]
    n_chunks = per_w // _CHUNK
    n_laps = n_chunks // _NBUF
    base_w = wid * per_w

    pltpu.sync_copy(item_idx.at[pl.ds(base_w, per_w)], idx_v)

    per_w_user = uidx_v.shape[0]
    pltpu.sync_copy(user_idx.at[pl.ds(wid * per_w_user, per_w_user)], uidx_v)
    pltpu.async_copy(table_user.at[uidx_v], urows_v, usem)

    def gather(j, b):
        return pltpu.make_async_copy(
            table_item.at[idx_v.at[pl.ds(j * _CHUNK, _CHUNK)]],
            rows_v.at[b],
            gsem.at[b],
        )

    def store(j, b):
        return pltpu.make_async_copy(
            rows_v.at[b],
            out_item.at[pl.ds(base_w + j * _CHUNK, _CHUNK)],
            osem.at[b],
        )

    for b in range(_NBUF):
        gather(b, b).start()

    def lap_body(lap, carry):
        jj = lap * _NBUF
        for b in range(_NBUF):
            gather(jj + b, b).wait()
            store(jj + b, b).start()
        for b in range(_NBUF):
            store(jj + b, b).wait()
            gather(jj + _NBUF + b, b).start()
        return carry

    lax.fori_loop(0, n_laps - 1, lap_body, 0)

    jj = (n_laps - 1) * _NBUF
    for b in range(_NBUF):
        gather(jj + b, b).wait()
        store(jj + b, b).start()
    for b in range(_NBUF):
        store(jj + b, b).wait()

    pltpu.make_async_copy(table_user.at[uidx_v], urows_v, usem).wait()
    pltpu.sync_copy(urows_v, out_user.at[pl.ds(wid * per_w_user, per_w_user)])


def kernel(values_item_hist, values_user_cat, table_item, table_user):
    n_hist = values_item_hist.shape[0]
    n_user = values_user_cat.shape[0]
    dim_item = table_item.shape[1]
    dim_user = table_user.shape[1]

    per_w = n_hist // _NW
    per_w_user = n_user // _NW

    mesh = plsc.VectorSubcoreMesh(core_axis_name="c", subcore_axis_name="s")
    f = pl.kernel(
        _emb_body,
        out_type=(
            jax.ShapeDtypeStruct((n_hist, dim_item), table_item.dtype),
            jax.ShapeDtypeStruct((n_user, dim_user), table_user.dtype),
        ),
        mesh=mesh,
        compiler_params=pltpu.CompilerParams(use_tc_tiling_on_sc=False),
        scratch_types=[
            pltpu.VMEM((per_w,), jnp.int32),
            pltpu.VMEM((_NBUF, _CHUNK, dim_item), jnp.float32),
            pltpu.VMEM((per_w_user,), jnp.int32),
            pltpu.VMEM((per_w_user, dim_user), jnp.float32),
            pltpu.SemaphoreType.DMA((_NBUF,)),
            pltpu.SemaphoreType.DMA((_NBUF,)),
            pltpu.SemaphoreType.DMA,
        ],
    )
    return f(values_item_hist, values_user_cat, table_item, table_user)
